# baseline (speedup 1.0000x reference)
import numpy as _np

import jax
import jax.numpy as jnp
from jax import lax
from jax.experimental import pallas as pl
from jax.experimental.pallas import tpu as pltpu
from jax.experimental.pallas import tpu_sc as plsc

N = 10000
D = 128
H = D // 2
E = 320000
NC = 2
NS = 16
EPW = E // NS
C = 80
B = 2000
CPB = B // C
NCHUNK = EPW // C
R = 4
RPW = 624
LANES = 16
HV = H // LANES

_ROWS = _np.arange(60)
_COMBO_I0 = _np.minimum(_ROWS // 12, 4).astype(_np.int32)
_COMBO_I1 = ((_ROWS % 12) // 2).astype(_np.int32)
_COMBO_I2 = (_ROWS % 2).astype(_np.int32)


def _sc_body(xa_hbm, xb_hbm, src_hbm, dst2_hbm, a0_hbm, a1_hbm, a2_hbm,
             e0a_hbm, e0b_hbm, e1a_hbm, e1b_hbm, e2a_hbm, e2b_hbm,
             parta_hbm, partb_hbm,
             acc, xsp, combo_sp,
             srcb, cib, dstb,
             xrows0, xrows1, xrows2, xrows3,
             erows0, erows1, erows2, erows3,
             sem_i,
             sem_x0, sem_x1, sem_x2, sem_x3,
             sem_e0, sem_e1, sem_e2, sem_e3,
             sem_s0, sem_s1, sem_s2, sem_s3):
    c = lax.axis_index("c")
    s = lax.axis_index("s")

    xrows = (xrows0, xrows1, xrows2, xrows3)
    erows = (erows0, erows1, erows2, erows3)
    sem_x = (sem_x0, sem_x1, sem_x2, sem_x3)
    sem_e = (sem_e0, sem_e1, sem_e2, sem_e3)
    sem_s = (sem_s0, sem_s1, sem_s2, sem_s3)

    r0 = s * RPW
    t0 = NS * RPW + s * 8

    zero = jnp.zeros((LANES,), jnp.float32)

    def zrow(r, carry):
        for j in range(HV):
            xrows0[r, pl.ds(j * LANES, LANES)] = zero
        return carry

    lax.fori_loop(0, C, zrow, 0)
    for t in range(RPW // C):
        pltpu.sync_copy(xrows0, acc.at[pl.ds(r0 + t * C, C)])
    pltpu.sync_copy(xrows0.at[pl.ds(0, RPW % C)],
                    acc.at[pl.ds(r0 + (RPW // C) * C, RPW % C)])

    @pl.when(s < 2)
    def _zero_tail():
        pltpu.sync_copy(xrows0.at[pl.ds(0, 8)], acc.at[pl.ds(t0, 8)])

    @pl.when(c == 0)
    def _stage_xa():
        pltpu.sync_copy(xa_hbm.at[pl.ds(r0, RPW)], xsp.at[pl.ds(r0, RPW)])

        @pl.when(s < 2)
        def _tail():
            pltpu.sync_copy(xa_hbm.at[pl.ds(t0, 8)], xsp.at[pl.ds(t0, 8)])

    @pl.when(c == 1)
    def _stage_xb():
        pltpu.sync_copy(xb_hbm.at[pl.ds(r0, RPW)], xsp.at[pl.ds(r0, RPW)])

        @pl.when(s < 2)
        def _tail():
            pltpu.sync_copy(xb_hbm.at[pl.ds(t0, 8)], xsp.at[pl.ds(t0, 8)])

    @pl.when(s == 0)
    def _build_combo():
        def stage_tables(ta_hbm, tb_hbm, tc_hbm):
            pltpu.sync_copy(ta_hbm, erows0.at[pl.ds(0, 5)])
            pltpu.sync_copy(tb_hbm, erows0.at[pl.ds(5, 6)])
            pltpu.sync_copy(tc_hbm, erows0.at[pl.ds(11, 2)])

        @pl.when(c == 0)
        def _tabs_a():
            stage_tables(e0a_hbm, e1a_hbm, e2a_hbm)

        @pl.when(c == 1)
        def _tabs_b():
            stage_tables(e0b_hbm, e1b_hbm, e2b_hbm)

        def combo_row(r, carry):
            ia = jnp.minimum(r // 12, 4)
            ib = 5 + (r % 12) // 2
            ic = 11 + r % 2
            for j in range(HV):
                sl = pl.ds(j * LANES, LANES)
                xrows0[r, sl] = (erows0[ia, sl] + erows0[ib, sl]
                                 + erows0[ic, sl])
            return carry

        lax.fori_loop(0, 60, combo_row, 0)
        pltpu.sync_copy(xrows0.at[pl.ds(0, 64)], combo_sp)

    plsc.subcore_barrier()

    ebase = s * EPW

    def refresh_block(k):
        b0 = ebase + k * C
        cpa = pltpu.async_copy(a0_hbm.at[pl.ds(b0, B)], cib, sem_i)
        cpb = pltpu.async_copy(a1_hbm.at[pl.ds(b0, B)], srcb, sem_i)
        cpd = pltpu.async_copy(
            dst2_hbm.at[pl.ds(s * (EPW // C) + k, CPB)],
            dstb.at[(k // CPB) % 2], sem_i)
        cpa.wait()
        cpb.wait()

        @plsc.parallel_loop(0, B // LANES, step=1, unroll=2)
        def _cirow(t):
            sl = pl.ds(t * LANES, LANES)
            cib[sl] = cib[sl] * 12 + srcb[sl] * 2

        pltpu.sync_copy(a2_hbm.at[pl.ds(b0, B)], srcb)

        @plsc.parallel_loop(0, B // LANES, step=1, unroll=2)
        def _cirow2(t):
            sl = pl.ds(t * LANES, LANES)
            cib[sl] = cib[sl] + srcb[sl]

        cpd.wait()
        pltpu.sync_copy(src_hbm.at[pl.ds(b0, B)], srcb)

    def start_gathers(k, p):
        off = (k % CPB) * C
        pltpu.async_copy(xsp.at[srcb.at[pl.ds(off, C)]], xrows[p], sem_x[p])
        pltpu.async_copy(combo_sp.at[cib.at[pl.ds(off, C)]], erows[p],
                         sem_e[p])

    def wait_gathers(p):
        pltpu.make_async_copy(xsp.at[srcb.at[pl.ds(0, C)]],
                              xrows[p], sem_x[p]).wait()
        pltpu.make_async_copy(combo_sp.at[cib.at[pl.ds(0, C)]],
                              erows[p], sem_e[p]).wait()

    def wait_scatter(q):
        pltpu.make_async_copy(xrows[q], acc.at[dstb.at[0, 0]],
                              sem_s[q]).wait()

    def chunk_body(k, pp):
        p1 = (pp + 1) % R
        p2 = (pp + 2) % R

        @pl.when(k % CPB == 0)
        def _refresh():
            refresh_block(k)

        @pl.when(k == 0)
        def _prime():
            start_gathers(0, pp)
            start_gathers(1, p1)

        @pl.when(jnp.logical_and(k > 0, k % CPB == 0))
        def _gather_here():
            start_gathers(k, pp)
            start_gathers(k + 1, p1)

        @pl.when(k < NCHUNK - 2)
        def _prefetch():
            @pl.when(k >= 2)
            def _drain():
                wait_scatter(p2)

            @pl.when((k + 2) % CPB >= 2)
            def _pref_gather():
                start_gathers(k + 2, p2)

        wait_gathers(pp)

        @plsc.parallel_loop(0, C, step=1, unroll=2)
        def _row(r):
            for j in range(HV):
                sl = pl.ds(j * LANES, LANES)
                xrows[pp][r, sl] = jnp.maximum(
                    xrows[pp][r, sl] + erows[pp][r, sl], 0.0)

        pltpu.async_copy(
            xrows[pp], acc.at[dstb.at[(k // CPB) % 2, k % CPB]],
            sem_s[pp], add=True)

    def pair(i, carry):
        k0 = i * 2

        @pl.when(i % 2 == 0)
        def _even():
            chunk_body(k0, 0)
            chunk_body(k0 + 1, 1)

        @pl.when(i % 2 == 1)
        def _odd():
            chunk_body(k0, 2)
            chunk_body(k0 + 1, 3)

        return carry

    lax.fori_loop(0, NCHUNK // 2, pair, 0)
    wait_scatter(0)
    wait_scatter(1)
    wait_scatter(2)
    wait_scatter(3)
    plsc.subcore_barrier()

    @pl.when(c == 0)
    def _out0():
        pltpu.sync_copy(acc.at[pl.ds(r0, RPW)], parta_hbm.at[pl.ds(r0, RPW)])

        @pl.when(s < 2)
        def _tail0():
            pltpu.sync_copy(acc.at[pl.ds(t0, 8)], parta_hbm.at[pl.ds(t0, 8)])

    @pl.when(c == 1)
    def _out1():
        pltpu.sync_copy(acc.at[pl.ds(r0, RPW)], partb_hbm.at[pl.ds(r0, RPW)])

        @pl.when(s < 2)
        def _tail1():
            pltpu.sync_copy(acc.at[pl.ds(t0, 8)], partb_hbm.at[pl.ds(t0, 8)])


_sc_segment = pl.kernel(
    _sc_body,
    out_type=(jax.ShapeDtypeStruct((N, H), jnp.float32),
              jax.ShapeDtypeStruct((N, H), jnp.float32)),
    mesh=plsc.VectorSubcoreMesh(core_axis_name="c", subcore_axis_name="s",
                                num_cores=NC, num_subcores=NS),
    compiler_params=pltpu.CompilerParams(use_tc_tiling_on_sc=False),
    scratch_types=[
        pltpu.VMEM_SHARED((N, H), jnp.float32),
        pltpu.VMEM_SHARED((N, H), jnp.float32),
        pltpu.VMEM_SHARED((64, H), jnp.float32),
        pltpu.VMEM((B,), jnp.int32),
        pltpu.VMEM((B,), jnp.int32),
        pltpu.VMEM((2, CPB, C), jnp.int32),
        pltpu.VMEM((C, H), jnp.float32),
        pltpu.VMEM((C, H), jnp.float32),
        pltpu.VMEM((C, H), jnp.float32),
        pltpu.VMEM((C, H), jnp.float32),
        pltpu.VMEM((C, H), jnp.float32),
        pltpu.VMEM((C, H), jnp.float32),
        pltpu.VMEM((C, H), jnp.float32),
        pltpu.VMEM((C, H), jnp.float32),
        pltpu.SemaphoreType.DMA,
        pltpu.SemaphoreType.DMA,
        pltpu.SemaphoreType.DMA,
        pltpu.SemaphoreType.DMA,
        pltpu.SemaphoreType.DMA,
        pltpu.SemaphoreType.DMA,
        pltpu.SemaphoreType.DMA,
        pltpu.SemaphoreType.DMA,
        pltpu.SemaphoreType.DMA,
        pltpu.SemaphoreType.DMA,
        pltpu.SemaphoreType.DMA,
        pltpu.SemaphoreType.DMA,
        pltpu.SemaphoreType.DMA,
    ],
)


def _mlp_body(x_ref, pa_ref, pb_ref, w1t_ref, b1_ref, gamma_ref, beta_ref,
              w2t_ref, b2_ref, eps_ref, out_ref):
    newx = jnp.concatenate([pa_ref[...], pb_ref[...]], axis=1)
    h = (1.0 + eps_ref[0, 0]) * x_ref[...] + newx
    h1 = jnp.dot(h, w1t_ref[...], preferred_element_type=jnp.float32) + b1_ref[...]
    mean = jnp.mean(h1, axis=0, keepdims=True)
    var = jnp.mean((h1 - mean) ** 2, axis=0, keepdims=True)
    hn = (h1 - mean) / jnp.sqrt(var + 1e-5) * gamma_ref[...] + beta_ref[...]
    h2 = jnp.maximum(hn, 0.0)
    out_ref[...] = (jnp.dot(h2, w2t_ref[...], preferred_element_type=jnp.float32)
                    + b2_ref[...])


_mlp = pl.pallas_call(
    _mlp_body,
    out_shape=jax.ShapeDtypeStruct((N, D), jnp.float32),
    in_specs=[pl.BlockSpec(memory_space=pltpu.VMEM)] * 9
    + [pl.BlockSpec(memory_space=pltpu.SMEM)],
    out_specs=pl.BlockSpec(memory_space=pltpu.VMEM),
)


@jax.jit
def kernel(x, edge_index, edge_attr, emb0, emb1, emb2,
           W1, b1, gamma, beta, W2, b2, eps):
    ei = edge_index.astype(jnp.int32)
    ea = edge_attr.astype(jnp.int32)
    parta, partb = _sc_segment(
        x[:, :H], x[:, H:], ei[0], ei[1].reshape(E // C, C),
        ea[:, 0], ea[:, 1], ea[:, 2],
        emb0[:, :H], emb0[:, H:], emb1[:, :H], emb1[:, H:],
        emb2[:, :H], emb2[:, H:])
    return _mlp(x, parta, partb,
                W1.T, b1.reshape(1, D), gamma.reshape(1, D),
                beta.reshape(1, D), W2.T, b2.reshape(1, D),
                eps.reshape(1, 1))


# speedup vs baseline: 1.0211x; 1.0211x over previous
import numpy as _np

import jax
import jax.numpy as jnp
from jax import lax
from jax.experimental import pallas as pl
from jax.experimental.pallas import tpu as pltpu
from jax.experimental.pallas import tpu_sc as plsc

N = 10000
D = 128
H = D // 2
E = 320000
NC = 2
NS = 16
EPW = E // NS
C = 80
B = 2000
CPB = B // C
NCHUNK = EPW // C
R = 4
RPW = 624
LANES = 16
HV = H // LANES

_ROWS = _np.arange(60)
_COMBO_I0 = _np.minimum(_ROWS // 12, 4).astype(_np.int32)
_COMBO_I1 = ((_ROWS % 12) // 2).astype(_np.int32)
_COMBO_I2 = (_ROWS % 2).astype(_np.int32)


def _sc_body(xa_hbm, xb_hbm, src_hbm, dst_hbm, a0_hbm, a1_hbm, a2_hbm,
             e0a_hbm, e0b_hbm, e1a_hbm, e1b_hbm, e2a_hbm, e2b_hbm,
             parta_hbm, partb_hbm,
             acc, xsp, combo_sp,
             srcb, cib, a2b,
             dst_v0, dst_v1, dst_v2, dst_v3,
             xrows0, xrows1, xrows2, xrows3,
             erows0, erows1, erows2, erows3,
             sem_i,
             sem_x0, sem_x1, sem_x2, sem_x3,
             sem_e0, sem_e1, sem_e2, sem_e3,
             sem_s0, sem_s1, sem_s2, sem_s3,
             sem_d0, sem_d1, sem_d2, sem_d3):
    c = lax.axis_index("c")
    s = lax.axis_index("s")

    dst_v = (dst_v0, dst_v1, dst_v2, dst_v3)
    xrows = (xrows0, xrows1, xrows2, xrows3)
    erows = (erows0, erows1, erows2, erows3)
    sem_x = (sem_x0, sem_x1, sem_x2, sem_x3)
    sem_e = (sem_e0, sem_e1, sem_e2, sem_e3)
    sem_s = (sem_s0, sem_s1, sem_s2, sem_s3)
    sem_d = (sem_d0, sem_d1, sem_d2, sem_d3)

    r0 = s * RPW
    t0 = NS * RPW + s * 8

    zero = jnp.zeros((LANES,), jnp.float32)

    def zrow(r, carry):
        for j in range(HV):
            xrows0[r, pl.ds(j * LANES, LANES)] = zero
        return carry

    lax.fori_loop(0, C, zrow, 0)
    for t in range(RPW // C):
        pltpu.sync_copy(xrows0, acc.at[pl.ds(r0 + t * C, C)])
    pltpu.sync_copy(xrows0.at[pl.ds(0, RPW % C)],
                    acc.at[pl.ds(r0 + (RPW // C) * C, RPW % C)])

    @pl.when(s < 2)
    def _zero_tail():
        pltpu.sync_copy(xrows0.at[pl.ds(0, 8)], acc.at[pl.ds(t0, 8)])

    @pl.when(c == 0)
    def _stage_xa():
        pltpu.sync_copy(xa_hbm.at[pl.ds(r0, RPW)], xsp.at[pl.ds(r0, RPW)])

        @pl.when(s < 2)
        def _tail():
            pltpu.sync_copy(xa_hbm.at[pl.ds(t0, 8)], xsp.at[pl.ds(t0, 8)])

    @pl.when(c == 1)
    def _stage_xb():
        pltpu.sync_copy(xb_hbm.at[pl.ds(r0, RPW)], xsp.at[pl.ds(r0, RPW)])

        @pl.when(s < 2)
        def _tail():
            pltpu.sync_copy(xb_hbm.at[pl.ds(t0, 8)], xsp.at[pl.ds(t0, 8)])

    @pl.when(s == 0)
    def _build_combo():
        def stage_tables(ta_hbm, tb_hbm, tc_hbm):
            pltpu.sync_copy(ta_hbm, erows0.at[pl.ds(0, 5)])
            pltpu.sync_copy(tb_hbm, erows0.at[pl.ds(5, 6)])
            pltpu.sync_copy(tc_hbm, erows0.at[pl.ds(11, 2)])

        @pl.when(c == 0)
        def _tabs_a():
            stage_tables(e0a_hbm, e1a_hbm, e2a_hbm)

        @pl.when(c == 1)
        def _tabs_b():
            stage_tables(e0b_hbm, e1b_hbm, e2b_hbm)

        def combo_row(r, carry):
            ia = jnp.minimum(r // 12, 4)
            ib = 5 + (r % 12) // 2
            ic = 11 + r % 2
            for j in range(HV):
                sl = pl.ds(j * LANES, LANES)
                xrows0[r, sl] = (erows0[ia, sl] + erows0[ib, sl]
                                 + erows0[ic, sl])
            return carry

        lax.fori_loop(0, 60, combo_row, 0)
        pltpu.sync_copy(xrows0.at[pl.ds(0, 64)], combo_sp)

    plsc.subcore_barrier()

    ebase = s * EPW

    def refresh_block(k):
        b0 = ebase + k * C
        cpa = pltpu.async_copy(a0_hbm.at[pl.ds(b0, B)], cib, sem_i)
        cpb = pltpu.async_copy(a1_hbm.at[pl.ds(b0, B)], srcb, sem_i)
        cpc = pltpu.async_copy(a2_hbm.at[pl.ds(b0, B)], a2b, sem_i)
        cpa.wait()
        cpb.wait()
        cpc.wait()

        @plsc.parallel_loop(0, B // LANES, step=1, unroll=2)
        def _cirow(t):
            sl = pl.ds(t * LANES, LANES)
            cib[sl] = cib[sl] * 12 + srcb[sl] * 2 + a2b[sl]

        pltpu.sync_copy(src_hbm.at[pl.ds(b0, B)], srcb)

    def start_gathers(k, p):
        off = (k % CPB) * C
        pltpu.async_copy(xsp.at[srcb.at[pl.ds(off, C)]], xrows[p], sem_x[p])
        pltpu.async_copy(combo_sp.at[cib.at[pl.ds(off, C)]], erows[p],
                         sem_e[p])

    def start_dst(k, p):
        pltpu.async_copy(dst_hbm.at[pl.ds(ebase + k * C, C)], dst_v[p],
                         sem_d[p])

    def wait_gathers(p):
        pltpu.make_async_copy(xsp.at[srcb.at[pl.ds(0, C)]],
                              xrows[p], sem_x[p]).wait()
        pltpu.make_async_copy(combo_sp.at[cib.at[pl.ds(0, C)]],
                              erows[p], sem_e[p]).wait()

    def wait_scatter(q):
        pltpu.make_async_copy(xrows[q], acc.at[dst_v[q]], sem_s[q]).wait()

    def wait_dst(p):
        pltpu.make_async_copy(dst_hbm.at[pl.ds(0, C)], dst_v[p],
                              sem_d[p]).wait()

    def chunk(k, carry):
        p = k % R

        @pl.when(k % CPB == 0)
        def _refresh():
            refresh_block(k)

        for pp in range(R):
            @pl.when(p == pp)
            def _pipe(pp=pp):
                p1 = (pp + 1) % R
                p2 = (pp + 2) % R

                @pl.when(k == 0)
                def _prime():
                    start_dst(0, pp)
                    start_dst(1, p1)
                    start_gathers(0, pp)
                    start_gathers(1, p1)

                @pl.when(jnp.logical_and(k > 0, k % CPB == 0))
                def _gather_here():
                    start_gathers(k, pp)
                    start_gathers(k + 1, p1)

                @pl.when(k < NCHUNK - 2)
                def _prefetch():
                    @pl.when(k >= 2)
                    def _drain():
                        wait_scatter(p2)

                    start_dst(k + 2, p2)

                    @pl.when((k + 2) % CPB >= 2)
                    def _pref_gather():
                        start_gathers(k + 2, p2)

                wait_gathers(pp)

                @plsc.parallel_loop(0, C, step=1, unroll=2)
                def _row(r):
                    for j in range(HV):
                        sl = pl.ds(j * LANES, LANES)
                        xrows[pp][r, sl] = jnp.maximum(
                            xrows[pp][r, sl] + erows[pp][r, sl], 0.0)

                wait_dst(pp)
                pltpu.async_copy(xrows[pp], acc.at[dst_v[pp]], sem_s[pp],
                                 add=True)

        return carry

    lax.fori_loop(0, NCHUNK, chunk, 0)
    wait_scatter(0)
    wait_scatter(1)
    wait_scatter(2)
    wait_scatter(3)
    plsc.subcore_barrier()

    @pl.when(c == 0)
    def _out0():
        pltpu.sync_copy(acc.at[pl.ds(r0, RPW)], parta_hbm.at[pl.ds(r0, RPW)])

        @pl.when(s < 2)
        def _tail0():
            pltpu.sync_copy(acc.at[pl.ds(t0, 8)], parta_hbm.at[pl.ds(t0, 8)])

    @pl.when(c == 1)
    def _out1():
        pltpu.sync_copy(acc.at[pl.ds(r0, RPW)], partb_hbm.at[pl.ds(r0, RPW)])

        @pl.when(s < 2)
        def _tail1():
            pltpu.sync_copy(acc.at[pl.ds(t0, 8)], partb_hbm.at[pl.ds(t0, 8)])


_sc_segment = pl.kernel(
    _sc_body,
    out_type=(jax.ShapeDtypeStruct((N, H), jnp.float32),
              jax.ShapeDtypeStruct((N, H), jnp.float32)),
    mesh=plsc.VectorSubcoreMesh(core_axis_name="c", subcore_axis_name="s",
                                num_cores=NC, num_subcores=NS),
    compiler_params=pltpu.CompilerParams(use_tc_tiling_on_sc=False),
    scratch_types=[
        pltpu.VMEM_SHARED((N, H), jnp.float32),
        pltpu.VMEM_SHARED((N, H), jnp.float32),
        pltpu.VMEM_SHARED((64, H), jnp.float32),
        pltpu.VMEM((B,), jnp.int32),
        pltpu.VMEM((B,), jnp.int32),
        pltpu.VMEM((B,), jnp.int32),
        pltpu.VMEM((C,), jnp.int32),
        pltpu.VMEM((C,), jnp.int32),
        pltpu.VMEM((C,), jnp.int32),
        pltpu.VMEM((C,), jnp.int32),
        pltpu.VMEM((C, H), jnp.float32),
        pltpu.VMEM((C, H), jnp.float32),
        pltpu.VMEM((C, H), jnp.float32),
        pltpu.VMEM((C, H), jnp.float32),
        pltpu.VMEM((C, H), jnp.float32),
        pltpu.VMEM((C, H), jnp.float32),
        pltpu.VMEM((C, H), jnp.float32),
        pltpu.VMEM((C, H), jnp.float32),
        pltpu.SemaphoreType.DMA,
        pltpu.SemaphoreType.DMA,
        pltpu.SemaphoreType.DMA,
        pltpu.SemaphoreType.DMA,
        pltpu.SemaphoreType.DMA,
        pltpu.SemaphoreType.DMA,
        pltpu.SemaphoreType.DMA,
        pltpu.SemaphoreType.DMA,
        pltpu.SemaphoreType.DMA,
        pltpu.SemaphoreType.DMA,
        pltpu.SemaphoreType.DMA,
        pltpu.SemaphoreType.DMA,
        pltpu.SemaphoreType.DMA,
        pltpu.SemaphoreType.DMA,
        pltpu.SemaphoreType.DMA,
        pltpu.SemaphoreType.DMA,
        pltpu.SemaphoreType.DMA,
    ],
)


def _mlp_body(x_ref, pa_ref, pb_ref, w1t_ref, b1_ref, gamma_ref, beta_ref,
              w2t_ref, b2_ref, eps_ref, out_ref):
    newx = jnp.concatenate([pa_ref[...], pb_ref[...]], axis=1)
    h = (1.0 + eps_ref[0, 0]) * x_ref[...] + newx
    h1 = jnp.dot(h, w1t_ref[...], preferred_element_type=jnp.float32) + b1_ref[...]
    mean = jnp.mean(h1, axis=0, keepdims=True)
    var = jnp.mean((h1 - mean) ** 2, axis=0, keepdims=True)
    hn = (h1 - mean) / jnp.sqrt(var + 1e-5) * gamma_ref[...] + beta_ref[...]
    h2 = jnp.maximum(hn, 0.0)
    out_ref[...] = (jnp.dot(h2, w2t_ref[...], preferred_element_type=jnp.float32)
                    + b2_ref[...])


_mlp = pl.pallas_call(
    _mlp_body,
    out_shape=jax.ShapeDtypeStruct((N, D), jnp.float32),
    in_specs=[pl.BlockSpec(memory_space=pltpu.VMEM)] * 9
    + [pl.BlockSpec(memory_space=pltpu.SMEM)],
    out_specs=pl.BlockSpec(memory_space=pltpu.VMEM),
)


@jax.jit
def kernel(x, edge_index, edge_attr, emb0, emb1, emb2,
           W1, b1, gamma, beta, W2, b2, eps):
    ei = edge_index.astype(jnp.int32)
    ea = edge_attr.astype(jnp.int32)
    parta, partb = _sc_segment(
        x[:, :H], x[:, H:], ei[0], ei[1], ea[:, 0], ea[:, 1], ea[:, 2],
        emb0[:, :H], emb0[:, H:], emb1[:, :H], emb1[:, H:],
        emb2[:, :H], emb2[:, H:])
    return _mlp(x, parta, partb,
                W1.T, b1.reshape(1, D), gamma.reshape(1, D),
                beta.reshape(1, D), W2.T, b2.reshape(1, D),
                eps.reshape(1, 1))
